# baseline (speedup 1.0000x reference)
import jax
import jax.numpy as jnp
from jax.experimental import pallas as pl
from jax.experimental.pallas import tpu as pltpu

_CHUNKS = 2


def _moe_kernel(x_ref, logits_ref, w13_ref, w2_ref, out_ref, coeff_ref):
    e = pl.program_id(0)
    c = pl.program_id(1)

    @pl.when((e == 0) & (c == 0))
    def _init():
        logits = logits_ref[...]
        num_experts = logits.shape[1]
        m1 = jnp.max(logits, axis=-1, keepdims=True)
        idx1 = jnp.argmax(logits, axis=-1, keepdims=True)
        neg = jnp.finfo(jnp.float32).min
        cols = jax.lax.broadcasted_iota(jnp.int32, logits.shape, 1)
        masked = jnp.where(cols == idx1, neg, logits)
        m2 = jnp.max(masked, axis=-1, keepdims=True)
        idx2 = jnp.argmax(masked, axis=-1, keepdims=True)
        r = jnp.exp(m2 - m1)
        w1 = 1.0 / (1.0 + r)
        w2c = r / (1.0 + r)
        coeff = jnp.where(cols == idx1, w1, 0.0) + jnp.where(cols == idx2, w2c, 0.0)
        for ei in range(num_experts):
            coeff_ref[ei] = coeff[:, ei : ei + 1]
        out_ref[...] = jnp.zeros_like(out_ref)

    x = x_ref[...]
    wg = w13_ref[0, 0]
    wu = w13_ref[0, 1]
    w2m = w2_ref[0]
    g = jax.lax.dot_general(
        x, wg, (((1,), (1,)), ((), ())), preferred_element_type=jnp.float32
    )
    u = jax.lax.dot_general(
        x, wu, (((1,), (1,)), ((), ())), preferred_element_type=jnp.float32
    )
    h = g * jax.nn.sigmoid(g) * u
    y = jax.lax.dot_general(
        h, w2m, (((1,), (1,)), ((), ())), preferred_element_type=jnp.float32
    )
    out_ref[...] += coeff_ref[e] * y


def kernel(hidden_states, router_logits, w13_weight, w2_weight):
    tokens, hidden = hidden_states.shape
    num_experts = w13_weight.shape[0]
    inter = w2_weight.shape[2]
    ic = inter // _CHUNKS
    w13v = w13_weight.reshape(num_experts, 2, inter, hidden)
    return pl.pallas_call(
        _moe_kernel,
        grid=(num_experts, _CHUNKS),
        in_specs=[
            pl.BlockSpec((tokens, hidden), lambda e, c: (0, 0)),
            pl.BlockSpec((tokens, num_experts), lambda e, c: (0, 0)),
            pl.BlockSpec((1, 2, ic, hidden), lambda e, c: (e, 0, c, 0)),
            pl.BlockSpec((1, hidden, ic), lambda e, c: (e, 0, c)),
        ],
        out_specs=pl.BlockSpec((tokens, hidden), lambda e, c: (0, 0)),
        out_shape=jax.ShapeDtypeStruct((tokens, hidden), jnp.float32),
        scratch_shapes=[pltpu.VMEM((num_experts, tokens, 1), jnp.float32)],
    )(hidden_states, router_logits, w13v, w2_weight)


# speedup vs baseline: 1.0683x; 1.0683x over previous
import jax
import jax.numpy as jnp
from jax.experimental import pallas as pl
from jax.experimental.pallas import tpu as pltpu


def _moe_kernel(x_ref, logits_ref, w13_ref, w2_ref, out_ref, coeff_ref):
    e = pl.program_id(0)

    @pl.when(e == 0)
    def _init():
        logits = logits_ref[...]
        num_experts = logits.shape[1]
        m1 = jnp.max(logits, axis=-1, keepdims=True)
        idx1 = jnp.argmax(logits, axis=-1, keepdims=True)
        neg = jnp.finfo(jnp.float32).min
        cols = jax.lax.broadcasted_iota(jnp.int32, logits.shape, 1)
        masked = jnp.where(cols == idx1, neg, logits)
        m2 = jnp.max(masked, axis=-1, keepdims=True)
        idx2 = jnp.argmax(masked, axis=-1, keepdims=True)
        r = jnp.exp(m2 - m1)
        w1 = 1.0 / (1.0 + r)
        w2c = r / (1.0 + r)
        coeff = jnp.where(cols == idx1, w1, 0.0) + jnp.where(cols == idx2, w2c, 0.0)
        for ei in range(num_experts):
            coeff_ref[ei] = coeff[:, ei : ei + 1]
        out_ref[...] = jnp.zeros_like(out_ref)

    x = x_ref[...]
    w13 = w13_ref[0]
    w2m = w2_ref[0]
    inter = w2m.shape[1]
    gate_up = jax.lax.dot_general(
        x, w13, (((1,), (1,)), ((), ())), preferred_element_type=jnp.float32
    )
    gate = gate_up[:, :inter]
    up = gate_up[:, inter:]
    h = gate * jax.nn.sigmoid(gate) * up
    y = jax.lax.dot_general(
        h, w2m, (((1,), (1,)), ((), ())), preferred_element_type=jnp.float32
    )
    out_ref[...] += coeff_ref[e] * y


def kernel(hidden_states, router_logits, w13_weight, w2_weight):
    tokens, hidden = hidden_states.shape
    num_experts = w13_weight.shape[0]
    inter = w2_weight.shape[2]
    return pl.pallas_call(
        _moe_kernel,
        grid=(num_experts,),
        in_specs=[
            pl.BlockSpec((tokens, hidden), lambda e: (0, 0)),
            pl.BlockSpec((tokens, num_experts), lambda e: (0, 0)),
            pl.BlockSpec((1, 2 * inter, hidden), lambda e: (e, 0, 0)),
            pl.BlockSpec((1, hidden, inter), lambda e: (e, 0, 0)),
        ],
        out_specs=pl.BlockSpec((tokens, hidden), lambda e: (0, 0)),
        out_shape=jax.ShapeDtypeStruct((tokens, hidden), jnp.float32),
        scratch_shapes=[pltpu.VMEM((num_experts, tokens, 1), jnp.float32)],
    )(hidden_states, router_logits, w13_weight, w2_weight)
